# baseline (speedup 1.0000x reference)
import jax
import jax.numpy as jnp
from jax import lax
from jax.experimental import pallas as pl
from jax.experimental.pallas import tpu as pltpu
from jax.experimental.pallas import tpu_sc as plsc

VOCAB = 1000000
EMB = 64
BATCH = 16384

NC = 2
NS = 16
NW = NC * NS
B_PER_W = BATCH // NW
CHUNK = 128
NCHUNK = B_PER_W // CHUNK

_MESH = plsc.VectorSubcoreMesh(core_axis_name="c", subcore_axis_name="s")


def _body(idx_hbm, table_hbm, out_hbm, idx_v, rows_v, sem):
    wid = lax.axis_index("s") * NC + lax.axis_index("c")
    pltpu.sync_copy(idx_hbm.at[pl.ds(wid * NCHUNK, NCHUNK)], idx_v)
    copies = [
        pltpu.async_copy(
            table_hbm.at[idx_v.at[j]],
            rows_v.at[pl.ds(j * CHUNK, CHUNK)],
            sem,
        )
        for j in range(NCHUNK)
    ]
    for cp in copies:
        cp.wait()
    pltpu.sync_copy(rows_v, out_hbm.at[pl.ds(wid * B_PER_W, B_PER_W)])


_gather = pl.kernel(
    _body,
    out_type=jax.ShapeDtypeStruct((BATCH, EMB), jnp.float32),
    mesh=_MESH,
    compiler_params=pltpu.CompilerParams(use_tc_tiling_on_sc=False),
    scratch_types=[
        pltpu.VMEM((NCHUNK, CHUNK), jnp.int32),
        pltpu.VMEM((B_PER_W, EMB), jnp.float32),
        pltpu.SemaphoreType.DMA,
    ],
)


@jax.jit
def kernel(inputs, embedding_table):
    idx2d = inputs.reshape(NW * NCHUNK, CHUNK)
    return _gather(idx2d, embedding_table)


# speedup vs baseline: 1.6320x; 1.6320x over previous
import jax
import jax.numpy as jnp
from jax import lax
from jax.experimental import pallas as pl
from jax.experimental.pallas import tpu as pltpu
from jax.experimental.pallas import tpu_sc as plsc

VOCAB = 1000000
EMB = 64
BATCH = 16384

NC = 2
NS = 16
NW = NC * NS
B_PER_W = BATCH // NW
CH = 16
NCH = B_PER_W // CH

_MESH = plsc.VectorSubcoreMesh(core_axis_name="c", subcore_axis_name="s")


def _body(idx_hbm, table_hbm, out_hbm, idx_v, rows_v, sem):
    wid = lax.axis_index("s") * NC + lax.axis_index("c")
    base = wid * B_PER_W
    pltpu.sync_copy(idx_hbm.at[pl.ds(base, B_PER_W)], idx_v)

    def chunk(k, carry):
        off = k * CH
        iv = idx_v[pl.ds(off, CH)]
        copies = []
        for r in range(CH):
            i = iv[r]
            copies.append(
                pltpu.async_copy(
                    table_hbm.at[pl.ds(i, 1)],
                    rows_v.at[pl.ds(off + r, 1)],
                    sem,
                )
            )
        for cp in copies:
            cp.wait()
        return carry

    lax.fori_loop(0, NCH, chunk, 0, unroll=False)
    pltpu.sync_copy(rows_v, out_hbm.at[pl.ds(base, B_PER_W)])


_gather = pl.kernel(
    _body,
    out_type=jax.ShapeDtypeStruct((BATCH, EMB), jnp.float32),
    mesh=_MESH,
    scratch_types=[
        pltpu.VMEM((B_PER_W,), jnp.int32),
        pltpu.VMEM((B_PER_W, EMB), jnp.float32),
        pltpu.SemaphoreType.DMA,
    ],
)


@jax.jit
def kernel(inputs, embedding_table):
    return _gather(inputs, embedding_table)


# speedup vs baseline: 1.7610x; 1.0790x over previous
import jax
import jax.numpy as jnp
from jax import lax
from jax.experimental import pallas as pl
from jax.experimental.pallas import tpu as pltpu
from jax.experimental.pallas import tpu_sc as plsc

VOCAB = 1000000
EMB = 64
BATCH = 16384

NC = 2
NS = 16
NW = NC * NS
SUBC = 512
NSUB = VOCAB // SUBC
TAIL = VOCAB - NSUB * SUBC
NCHUNK_PER_W = (NSUB + NW - 1) // NW
LANES = 16
SEL_CAP = BATCH + 2 * LANES
DUMP = SEL_CAP - 1
SENTINEL = 1 << 30


def _match01(chunk_vec, target):
    d = chunk_vec ^ target
    return ((d - 1) >> 31) & 1


_MESH = plsc.VectorSubcoreMesh(core_axis_name="c", subcore_axis_name="s")


def _body(idx_hbm, table_t_hbm, out_hbm, all_idx, sel_idx, sel_pos, buf,
          tailbuf, ring, mcnt, rowsem):
    wid = lax.axis_index("s") * NC + lax.axis_index("c")
    mcnt[0] = 0

    pltpu.sync_copy(idx_hbm, all_idx)

    lane_iota = lax.iota(jnp.int32, LANES)

    def lane_gather(x, idx):
        return lax.gather(
            x, idx[:, None],
            dimension_numbers=lax.GatherDimensionNumbers(
                offset_dims=(), collapsed_slice_dims=(0,),
                start_index_map=(0,)),
            slice_sizes=(1,),
            mode=lax.GatherScatterMode.PROMISE_IN_BOUNDS,
        )

    def prefix16(x):
        for s in (1, 2, 4, 8):
            shifted = lane_gather(x, jnp.maximum(lane_iota - s, 0))
            keep = 1 - (((lane_iota - s) >> 31) & 1)
            x = x + shifted * keep
        return x

    def scan(k, cnt):
        iv = all_idx[pl.ds(k * LANES, LANES)]
        e = _match01((iv >> 9) & (NW - 1), wid)
        incl = prefix16(e)
        tgt = cnt + incl - 1
        tgt = e * tgt + (1 - e) * DUMP
        plsc.store_scatter(sel_idx, [tgt], iv)
        plsc.store_scatter(sel_pos, [tgt], lane_iota + k * LANES)
        return cnt + incl[LANES - 1]

    cnt = lax.fori_loop(0, BATCH // LANES, scan, jnp.int32(0), unroll=False)
    sel_idx[pl.ds(cnt, LANES)] = jnp.full((LANES,), SENTINEL, jnp.int32)
    ngrp = (cnt + LANES - 1) // LANES

    def extract_rows(c, src_buf):
        def group(u, carry):
            sv = sel_idx[pl.ds(u * LANES, LANES)]
            cv = sv >> 9
            e2 = _match01(cv, c)

            @pl.when(jnp.any(cv == c))
            def _():
                pv = sel_pos[pl.ds(u * LANES, LANES)]
                for r in range(LANES):
                    @pl.when(cv[r] == c)
                    def _():
                        col = sv[r] & (SUBC - 1)
                        mc = mcnt[0]
                        slot = mc & 15

                        @pl.when(mc >= 16)
                        def _():
                            pltpu.make_async_copy(
                                out_hbm.at[pl.ds(0, EMB)],
                                ring.at[pl.ds(0, EMB)],
                                rowsem,
                            ).wait()

                        for g in range(EMB // LANES):
                            vals = plsc.load_gather(
                                src_buf,
                                [lane_iota + g * LANES,
                                 jnp.full((LANES,), col, jnp.int32)])
                            ring[pl.ds(slot * EMB + g * LANES, LANES)] = vals
                        pos = pv[r]
                        pltpu.async_copy(
                            ring.at[pl.ds(slot * EMB, EMB)],
                            out_hbm.at[pl.ds(pl.multiple_of(pos * EMB, EMB),
                                             EMB)],
                            rowsem,
                        )
                        mcnt[0] = mc + 1
            return carry

        lax.fori_loop(0, ngrp, group, 0, unroll=False)

    def per_chunk(t, carry):
        c = wid + t * NW

        @pl.when(c < NSUB)
        def _():
            pltpu.sync_copy(
                table_t_hbm.at[:, pl.ds(pl.multiple_of(c * SUBC, 128),
                                        SUBC)],
                buf,
            )
            extract_rows(c, buf)
        return carry

    lax.fori_loop(0, NCHUNK_PER_W, per_chunk, 0, unroll=False)

    @pl.when(wid == NSUB % NW)
    def _():
        pltpu.sync_copy(table_t_hbm.at[:, pl.ds(NSUB * SUBC, TAIL)], tailbuf)
        extract_rows(jnp.int32(NSUB), tailbuf)

    def drain(r, carry):
        @pl.when(r < jnp.minimum(mcnt[0], 16))
        def _():
            pltpu.make_async_copy(
                out_hbm.at[pl.ds(0, EMB)],
                ring.at[pl.ds(0, EMB)],
                rowsem,
            ).wait()
        return carry

    lax.fori_loop(0, 16, drain, 0, unroll=False)


_gather = pl.kernel(
    _body,
    out_type=jax.ShapeDtypeStruct((BATCH * EMB,), jnp.float32),
    mesh=_MESH,
    compiler_params=pltpu.CompilerParams(needs_layout_passes=False),
    scratch_types=[
        pltpu.VMEM((BATCH,), jnp.int32),
        pltpu.VMEM((SEL_CAP,), jnp.int32),
        pltpu.VMEM((SEL_CAP,), jnp.int32),
        pltpu.VMEM((EMB, SUBC), jnp.float32),
        pltpu.VMEM((EMB, TAIL), jnp.float32),
        pltpu.VMEM((16 * EMB,), jnp.float32),
        pltpu.SMEM((1,), jnp.int32),
        pltpu.SemaphoreType.DMA,
    ],
)


@jax.jit
def kernel(inputs, embedding_table):
    flat = _gather(inputs, embedding_table.T)
    return flat.reshape(BATCH, EMB)


# speedup vs baseline: 2.2908x; 1.3009x over previous
import jax
import jax.numpy as jnp
from jax import lax
from jax.experimental import pallas as pl
from jax.experimental.pallas import tpu as pltpu
from jax.experimental.pallas import tpu_sc as plsc

VOCAB = 1000000
EMB = 64
BATCH = 16384

NC = 2
NS = 16
NW = NC * NS
SUBC = 512
NSUB = VOCAB // SUBC
TAIL = VOCAB - NSUB * SUBC
NCHUNK_PER_W = (NSUB + NW - 1) // NW
LANES = 16
SEL_CAP = BATCH + 2 * LANES
DUMP = SEL_CAP - 1
SENTINEL = 1 << 30


def _match01(chunk_vec, target):
    d = chunk_vec ^ target
    return ((d - 1) >> 31) & 1


_MESH = plsc.VectorSubcoreMesh(core_axis_name="c", subcore_axis_name="s")


def _body(idx_hbm, table_t_hbm, out_hbm, all_idx, sel_idx, sel_pos, buf0,
          buf1, tailbuf, ring, mcnt, sem0, sem1, rowsem):
    wid = lax.axis_index("s") * NC + lax.axis_index("c")
    mcnt[0] = 0

    pltpu.sync_copy(idx_hbm, all_idx)

    lane_iota = lax.iota(jnp.int32, LANES)

    def lane_gather(x, idx):
        return lax.gather(
            x, idx[:, None],
            dimension_numbers=lax.GatherDimensionNumbers(
                offset_dims=(), collapsed_slice_dims=(0,),
                start_index_map=(0,)),
            slice_sizes=(1,),
            mode=lax.GatherScatterMode.PROMISE_IN_BOUNDS,
        )

    def prefix16(x):
        for s in (1, 2, 4, 8):
            shifted = lane_gather(x, jnp.maximum(lane_iota - s, 0))
            keep = 1 - (((lane_iota - s) >> 31) & 1)
            x = x + shifted * keep
        return x

    mcnt[1] = 0

    def scan(k, carry):
        iv = all_idx[pl.ds(k * LANES, LANES)]
        ov = (iv >> 9) & (NW - 1)

        @pl.when(jnp.any(ov == wid))
        def _():
            cnt = mcnt[1]
            e = _match01(ov, wid)
            incl = prefix16(e)
            tgt = cnt + incl - 1
            tgt = e * tgt + (1 - e) * DUMP
            plsc.store_scatter(sel_idx, [tgt], iv)
            plsc.store_scatter(sel_pos, [tgt], lane_iota + k * LANES)
            mcnt[1] = cnt + incl[LANES - 1]
        return carry

    lax.fori_loop(0, BATCH // LANES, scan, 0, unroll=False)
    cnt = mcnt[1]
    sel_idx[pl.ds(cnt, LANES)] = jnp.full((LANES,), SENTINEL, jnp.int32)
    ngrp = (cnt + LANES - 1) // LANES

    def extract_rows(c, src_buf):
        def group(u, carry):
            sv = sel_idx[pl.ds(u * LANES, LANES)]
            cv = sv >> 9

            @pl.when(jnp.any(cv == c))
            def _():
                pv = sel_pos[pl.ds(u * LANES, LANES)]
                for r in range(LANES):
                    @pl.when(cv[r] == c)
                    def _():
                        col = sv[r] & (SUBC - 1)
                        mc = mcnt[0]
                        slot = mc & 15

                        @pl.when(mc >= 16)
                        def _():
                            pltpu.make_async_copy(
                                out_hbm.at[pl.ds(0, EMB)],
                                ring.at[pl.ds(0, EMB)],
                                rowsem,
                            ).wait()

                        for g in range(EMB // LANES):
                            vals = plsc.load_gather(
                                src_buf,
                                [lane_iota + g * LANES,
                                 jnp.full((LANES,), col, jnp.int32)])
                            ring[pl.ds(slot * EMB + g * LANES, LANES)] = vals
                        pos = pv[r]
                        pltpu.async_copy(
                            ring.at[pl.ds(slot * EMB, EMB)],
                            out_hbm.at[pl.ds(pl.multiple_of(pos * EMB, EMB),
                                             EMB)],
                            rowsem,
                        )
                        mcnt[0] = mc + 1
            return carry

        lax.fori_loop(0, ngrp, group, 0, unroll=False)

    def fetch(c, dst, sem):
        pltpu.async_copy(
            table_t_hbm.at[:, pl.ds(pl.multiple_of(c * SUBC, 128), SUBC)],
            dst, sem)

    def bufwait(dst, sem):
        pltpu.make_async_copy(
            table_t_hbm.at[:, pl.ds(0, SUBC)], dst, sem).wait()

    @pl.when(wid < NSUB)
    def _():
        fetch(wid, buf0, sem0)

    def pair(p, carry):
        ca = wid + (2 * p) * NW
        cb = ca + NW
        cc = cb + NW

        @pl.when(ca < NSUB)
        def _():
            bufwait(buf0, sem0)

            @pl.when(cb < NSUB)
            def _():
                fetch(cb, buf1, sem1)
            extract_rows(ca, buf0)

        @pl.when(cb < NSUB)
        def _():
            bufwait(buf1, sem1)

            @pl.when(cc < NSUB)
            def _():
                fetch(cc, buf0, sem0)
            extract_rows(cb, buf1)
        return carry

    lax.fori_loop(0, (NCHUNK_PER_W + 1) // 2, pair, 0, unroll=False)

    @pl.when(wid == NSUB % NW)
    def _():
        pltpu.sync_copy(table_t_hbm.at[:, pl.ds(NSUB * SUBC, TAIL)], tailbuf)
        extract_rows(jnp.int32(NSUB), tailbuf)

    def drain(r, carry):
        @pl.when(r < jnp.minimum(mcnt[0], 16))
        def _():
            pltpu.make_async_copy(
                out_hbm.at[pl.ds(0, EMB)],
                ring.at[pl.ds(0, EMB)],
                rowsem,
            ).wait()
        return carry

    lax.fori_loop(0, 16, drain, 0, unroll=False)


_gather = pl.kernel(
    _body,
    out_type=jax.ShapeDtypeStruct((BATCH * EMB,), jnp.float32),
    mesh=_MESH,
    compiler_params=pltpu.CompilerParams(needs_layout_passes=False),
    scratch_types=[
        pltpu.VMEM((BATCH,), jnp.int32),
        pltpu.VMEM((SEL_CAP,), jnp.int32),
        pltpu.VMEM((SEL_CAP,), jnp.int32),
        pltpu.VMEM((EMB, SUBC), jnp.float32),
        pltpu.VMEM((EMB, SUBC), jnp.float32),
        pltpu.VMEM((EMB, TAIL), jnp.float32),
        pltpu.VMEM((16 * EMB,), jnp.float32),
        pltpu.SMEM((2,), jnp.int32),
        pltpu.SemaphoreType.DMA,
        pltpu.SemaphoreType.DMA,
        pltpu.SemaphoreType.DMA,
    ],
)


@jax.jit
def kernel(inputs, embedding_table):
    flat = _gather(inputs, embedding_table.T)
    return flat.reshape(BATCH, EMB)


# speedup vs baseline: 2.4005x; 1.0479x over previous
import jax
import jax.numpy as jnp
from jax import lax
from jax.experimental import pallas as pl
from jax.experimental.pallas import tpu as pltpu
from jax.experimental.pallas import tpu_sc as plsc

VOCAB = 1000000
EMB = 64
BATCH = 16384

NC = 2
NS = 16
NW = NC * NS
SUBC = 512
NSUB = VOCAB // SUBC
TAIL = VOCAB - NSUB * SUBC
NCHUNK_PER_W = (NSUB + NW - 1) // NW
LANES = 16
SEL_CAP = BATCH + 2 * LANES
DUMP = SEL_CAP - 1
SENTINEL = 1 << 30


def _match01(chunk_vec, target):
    d = chunk_vec ^ target
    return ((d - 1) >> 31) & 1


_MESH = plsc.VectorSubcoreMesh(core_axis_name="c", subcore_axis_name="s")


def _body(idx_hbm, table_t_hbm, out_hbm, all_idx, sel_idx, sel_pos, buf0,
          buf1, tailbuf, ring, mcnt, sem0, sem1, rowsem):
    wid = lax.axis_index("s") * NC + lax.axis_index("c")
    mcnt[0] = 0

    pltpu.sync_copy(idx_hbm, all_idx)

    lane_iota = lax.iota(jnp.int32, LANES)

    def lane_gather(x, idx):
        return lax.gather(
            x, idx[:, None],
            dimension_numbers=lax.GatherDimensionNumbers(
                offset_dims=(), collapsed_slice_dims=(0,),
                start_index_map=(0,)),
            slice_sizes=(1,),
            mode=lax.GatherScatterMode.PROMISE_IN_BOUNDS,
        )

    def prefix16(x):
        for s in (1, 2, 4, 8):
            shifted = lane_gather(x, jnp.maximum(lane_iota - s, 0))
            keep = 1 - (((lane_iota - s) >> 31) & 1)
            x = x + shifted * keep
        return x

    mcnt[1] = 0

    def scan(k, carry):
        iv = all_idx[pl.ds(k * LANES, LANES)]
        ov = (iv >> 9) & (NW - 1)

        @pl.when(plsc.all_reduce_population_count(ov == wid)[0] > 0)
        def _():
            cnt = mcnt[1]
            e = _match01(ov, wid)
            incl = prefix16(e)
            tgt = cnt + incl - 1
            tgt = e * tgt + (1 - e) * DUMP
            plsc.store_scatter(sel_idx, [tgt], iv)
            plsc.store_scatter(sel_pos, [tgt], lane_iota + k * LANES)
            mcnt[1] = cnt + incl[LANES - 1]
        return carry

    lax.fori_loop(0, BATCH // LANES, scan, 0, unroll=False)
    cnt = mcnt[1]
    sel_idx[pl.ds(cnt, LANES)] = jnp.full((LANES,), SENTINEL, jnp.int32)
    ngrp = (cnt + LANES - 1) // LANES

    def extract_rows(c, src_buf):
        def group(u, carry):
            sv = sel_idx[pl.ds(u * LANES, LANES)]
            cv = sv >> 9

            @pl.when(plsc.all_reduce_population_count(cv == c)[0] > 0)
            def _():
                pv = sel_pos[pl.ds(u * LANES, LANES)]
                for r in range(LANES):
                    @pl.when(cv[r] == c)
                    def _():
                        col = sv[r] & (SUBC - 1)
                        mc = mcnt[0]
                        slot = mc & 15

                        @pl.when(mc >= 16)
                        def _():
                            pltpu.make_async_copy(
                                out_hbm.at[pl.ds(0, EMB)],
                                ring.at[pl.ds(0, EMB)],
                                rowsem,
                            ).wait()

                        for g in range(EMB // LANES):
                            vals = plsc.load_gather(
                                src_buf,
                                [lane_iota + g * LANES,
                                 jnp.full((LANES,), col, jnp.int32)])
                            ring[pl.ds(slot * EMB + g * LANES, LANES)] = vals
                        pos = pv[r]
                        pltpu.async_copy(
                            ring.at[pl.ds(slot * EMB, EMB)],
                            out_hbm.at[pl.ds(pl.multiple_of(pos * EMB, EMB),
                                             EMB)],
                            rowsem,
                        )
                        mcnt[0] = mc + 1
            return carry

        lax.fori_loop(0, ngrp, group, 0, unroll=False)

    def fetch(c, dst, sem):
        pltpu.async_copy(
            table_t_hbm.at[:, pl.ds(pl.multiple_of(c * SUBC, 128), SUBC)],
            dst, sem)

    def bufwait(dst, sem):
        pltpu.make_async_copy(
            table_t_hbm.at[:, pl.ds(0, SUBC)], dst, sem).wait()

    @pl.when(wid < NSUB)
    def _():
        fetch(wid, buf0, sem0)

    def pair(p, carry):
        ca = wid + (2 * p) * NW
        cb = ca + NW
        cc = cb + NW

        @pl.when(ca < NSUB)
        def _():
            bufwait(buf0, sem0)

            @pl.when(cb < NSUB)
            def _():
                fetch(cb, buf1, sem1)
            extract_rows(ca, buf0)

        @pl.when(cb < NSUB)
        def _():
            bufwait(buf1, sem1)

            @pl.when(cc < NSUB)
            def _():
                fetch(cc, buf0, sem0)
            extract_rows(cb, buf1)
        return carry

    lax.fori_loop(0, (NCHUNK_PER_W + 1) // 2, pair, 0, unroll=False)

    @pl.when(wid == NSUB % NW)
    def _():
        pltpu.sync_copy(table_t_hbm.at[:, pl.ds(NSUB * SUBC, TAIL)], tailbuf)
        extract_rows(jnp.int32(NSUB), tailbuf)

    def drain(r, carry):
        @pl.when(r < jnp.minimum(mcnt[0], 16))
        def _():
            pltpu.make_async_copy(
                out_hbm.at[pl.ds(0, EMB)],
                ring.at[pl.ds(0, EMB)],
                rowsem,
            ).wait()
        return carry

    lax.fori_loop(0, 16, drain, 0, unroll=False)


_gather = pl.kernel(
    _body,
    out_type=jax.ShapeDtypeStruct((BATCH * EMB,), jnp.float32),
    mesh=_MESH,
    compiler_params=pltpu.CompilerParams(needs_layout_passes=False),
    scratch_types=[
        pltpu.VMEM((BATCH,), jnp.int32),
        pltpu.VMEM((SEL_CAP,), jnp.int32),
        pltpu.VMEM((SEL_CAP,), jnp.int32),
        pltpu.VMEM((EMB, SUBC), jnp.float32),
        pltpu.VMEM((EMB, SUBC), jnp.float32),
        pltpu.VMEM((EMB, TAIL), jnp.float32),
        pltpu.VMEM((16 * EMB,), jnp.float32),
        pltpu.SMEM((2,), jnp.int32),
        pltpu.SemaphoreType.DMA,
        pltpu.SemaphoreType.DMA,
        pltpu.SemaphoreType.DMA,
    ],
)


@jax.jit
def kernel(inputs, embedding_table):
    flat = _gather(inputs, embedding_table.T)
    return flat.reshape(BATCH, EMB)


# speedup vs baseline: 3.1039x; 1.2930x over previous
import jax
import jax.numpy as jnp
from jax import lax
from jax.experimental import pallas as pl
from jax.experimental.pallas import tpu as pltpu
from jax.experimental.pallas import tpu_sc as plsc

VOCAB = 1000000
EMB = 64
BATCH = 16384

NC = 2
NS = 16
NW = NC * NS
SUBC = 512
NSUB = VOCAB // SUBC
TAIL = VOCAB - NSUB * SUBC
NCHUNK_PER_W = (NSUB + NW - 1) // NW
LANES = 16
SEL_CAP = BATCH + 2 * LANES
DUMP = SEL_CAP - 1
SENTINEL = 1 << 30


def _match01(chunk_vec, target):
    d = chunk_vec ^ target
    return ((d - 1) >> 31) & 1


_MESH = plsc.VectorSubcoreMesh(core_axis_name="c", subcore_axis_name="s")


def _body(idx_hbm, table_t_hbm, out_hbm, all_idx, sel_idx, sel_pos, buf0,
          buf1, tailbuf, ring, mcnt, sem0, sem1, rowsem):
    wid = lax.axis_index("s") * NC + lax.axis_index("c")
    mcnt[0] = 0

    pltpu.sync_copy(idx_hbm, all_idx)

    lane_iota = lax.iota(jnp.int32, LANES)

    def lane_gather(x, idx):
        return lax.gather(
            x, idx[:, None],
            dimension_numbers=lax.GatherDimensionNumbers(
                offset_dims=(), collapsed_slice_dims=(0,),
                start_index_map=(0,)),
            slice_sizes=(1,),
            mode=lax.GatherScatterMode.PROMISE_IN_BOUNDS,
        )

    def prefix16(x):
        for s in (1, 2, 4, 8):
            shifted = lane_gather(x, jnp.maximum(lane_iota - s, 0))
            keep = 1 - (((lane_iota - s) >> 31) & 1)
            x = x + shifted * keep
        return x

    mcnt[1] = 0

    def scan(k, carry):
        iv = all_idx[pl.ds(k * LANES, LANES)]
        ov = (iv >> 9) & (NW - 1)

        @pl.when(plsc.all_reduce_population_count(ov == wid)[0] > 0)
        def _():
            cnt = mcnt[1]
            e = _match01(ov, wid)
            incl = prefix16(e)
            tgt = cnt + incl - 1
            tgt = e * tgt + (1 - e) * DUMP
            plsc.store_scatter(sel_idx, [tgt], iv)
            plsc.store_scatter(sel_pos, [tgt], lane_iota + k * LANES)
            mcnt[1] = cnt + incl[LANES - 1]
        return carry

    lax.fori_loop(0, BATCH // LANES, scan, 0, unroll=False)
    cnt = mcnt[1]
    sel_idx[pl.ds(cnt, LANES)] = jnp.full((LANES,), SENTINEL, jnp.int32)
    ngrp = (cnt + LANES - 1) // LANES

    def extract_rows(c, src_buf):
        def group(u, carry):
            sv = sel_idx[pl.ds(u * LANES, LANES)]
            cv = sv >> 9
            e2 = _match01(cv, c)
            f0 = plsc.all_reduce_ffs(e2 == 1)[0]

            def has_match(st):
                return st[1] < LANES

            def per_match(st):
                e2c, f = st
                fv = jnp.full((LANES,), 0, jnp.int32) + f
                pv = sel_pos[pl.ds(u * LANES, LANES)]
                col = lane_gather(sv, fv)[0] & (SUBC - 1)
                pos = lane_gather(pv, fv)[0]
                mc = mcnt[0]
                slot = mc & 15

                @pl.when(mc >= 16)
                def _():
                    pltpu.make_async_copy(
                        out_hbm.at[pl.ds(0, EMB)],
                        ring.at[pl.ds(0, EMB)],
                        rowsem,
                    ).wait()

                for g in range(EMB // LANES):
                    vals = plsc.load_gather(
                        src_buf,
                        [lane_iota + g * LANES,
                         jnp.full((LANES,), col, jnp.int32)])
                    ring[pl.ds(slot * EMB + g * LANES, LANES)] = vals
                pltpu.async_copy(
                    ring.at[pl.ds(slot * EMB, EMB)],
                    out_hbm.at[pl.ds(pl.multiple_of(pos * EMB, EMB), EMB)],
                    rowsem,
                )
                mcnt[0] = mc + 1
                e2n = e2c * (1 - _match01(lane_iota, f))
                fn = plsc.all_reduce_ffs(e2n == 1)[0]
                return (e2n, fn)

            lax.while_loop(has_match, per_match, (e2, f0))
            return carry

        lax.fori_loop(0, ngrp, group, 0, unroll=False)

    def fetch(c, dst, sem):
        pltpu.async_copy(
            table_t_hbm.at[:, pl.ds(pl.multiple_of(c * SUBC, 128), SUBC)],
            dst, sem)

    def bufwait(dst, sem):
        pltpu.make_async_copy(
            table_t_hbm.at[:, pl.ds(0, SUBC)], dst, sem).wait()

    @pl.when(wid < NSUB)
    def _():
        fetch(wid, buf0, sem0)

    def pair(p, carry):
        ca = wid + (2 * p) * NW
        cb = ca + NW
        cc = cb + NW

        @pl.when(ca < NSUB)
        def _():
            bufwait(buf0, sem0)

            @pl.when(cb < NSUB)
            def _():
                fetch(cb, buf1, sem1)
            extract_rows(ca, buf0)

        @pl.when(cb < NSUB)
        def _():
            bufwait(buf1, sem1)

            @pl.when(cc < NSUB)
            def _():
                fetch(cc, buf0, sem0)
            extract_rows(cb, buf1)
        return carry

    lax.fori_loop(0, (NCHUNK_PER_W + 1) // 2, pair, 0, unroll=False)

    @pl.when(wid == NSUB % NW)
    def _():
        pltpu.sync_copy(table_t_hbm.at[:, pl.ds(NSUB * SUBC, TAIL)], tailbuf)
        extract_rows(jnp.int32(NSUB), tailbuf)

    def drain(r, carry):
        @pl.when(r < jnp.minimum(mcnt[0], 16))
        def _():
            pltpu.make_async_copy(
                out_hbm.at[pl.ds(0, EMB)],
                ring.at[pl.ds(0, EMB)],
                rowsem,
            ).wait()
        return carry

    lax.fori_loop(0, 16, drain, 0, unroll=False)


_gather = pl.kernel(
    _body,
    out_type=jax.ShapeDtypeStruct((BATCH * EMB,), jnp.float32),
    mesh=_MESH,
    compiler_params=pltpu.CompilerParams(needs_layout_passes=False),
    scratch_types=[
        pltpu.VMEM((BATCH,), jnp.int32),
        pltpu.VMEM((SEL_CAP,), jnp.int32),
        pltpu.VMEM((SEL_CAP,), jnp.int32),
        pltpu.VMEM((EMB, SUBC), jnp.float32),
        pltpu.VMEM((EMB, SUBC), jnp.float32),
        pltpu.VMEM((EMB, TAIL), jnp.float32),
        pltpu.VMEM((16 * EMB,), jnp.float32),
        pltpu.SMEM((2,), jnp.int32),
        pltpu.SemaphoreType.DMA,
        pltpu.SemaphoreType.DMA,
        pltpu.SemaphoreType.DMA,
    ],
)


@jax.jit
def kernel(inputs, embedding_table):
    flat = _gather(inputs, embedding_table.T)
    return flat.reshape(BATCH, EMB)


# speedup vs baseline: 3.1569x; 1.0171x over previous
import jax
import jax.numpy as jnp
from jax import lax
from jax.experimental import pallas as pl
from jax.experimental.pallas import tpu as pltpu
from jax.experimental.pallas import tpu_sc as plsc

VOCAB = 1000000
EMB = 64
BATCH = 16384

NC = 2
NS = 16
NW = NC * NS
SUBC = 512
NSUB = VOCAB // SUBC
TAIL = VOCAB - NSUB * SUBC
NCHUNK_PER_W = (NSUB + NW - 1) // NW
LANES = 16
SEL_CAP = BATCH + 2 * LANES
DUMP = SEL_CAP - 1
SENTINEL = 1 << 30


def _match01(chunk_vec, target):
    d = chunk_vec ^ target
    return ((d - 1) >> 31) & 1


_MESH = plsc.VectorSubcoreMesh(core_axis_name="c", subcore_axis_name="s")


def _body(idx_hbm, table_t_hbm, out_hbm, all_idx, sel_idx, sel_pos, buf0,
          buf1, tailbuf, ring, mcnt, sem0, sem1, rowsem):
    wid = lax.axis_index("s") * NC + lax.axis_index("c")
    mcnt[0] = 0

    pltpu.sync_copy(idx_hbm, all_idx)

    lane_iota = lax.iota(jnp.int32, LANES)

    def lane_gather(x, idx):
        return lax.gather(
            x, idx[:, None],
            dimension_numbers=lax.GatherDimensionNumbers(
                offset_dims=(), collapsed_slice_dims=(0,),
                start_index_map=(0,)),
            slice_sizes=(1,),
            mode=lax.GatherScatterMode.PROMISE_IN_BOUNDS,
        )

    def prefix16(x):
        for s in (1, 2, 4, 8):
            shifted = lane_gather(x, jnp.maximum(lane_iota - s, 0))
            keep = 1 - (((lane_iota - s) >> 31) & 1)
            x = x + shifted * keep
        return x

    mcnt[1] = 0

    def scan(k, carry):
        iv = all_idx[pl.ds(k * LANES, LANES)]
        ov = (iv >> 9) & (NW - 1)

        @pl.when(plsc.all_reduce_population_count(ov == wid)[0] > 0)
        def _():
            cnt = mcnt[1]
            e = _match01(ov, wid)
            incl = prefix16(e)
            tgt = cnt + incl - 1
            tgt = e * tgt + (1 - e) * DUMP
            plsc.store_scatter(sel_idx, [tgt], iv)
            plsc.store_scatter(sel_pos, [tgt], lane_iota + k * LANES)
            mcnt[1] = cnt + incl[LANES - 1]
        return carry

    lax.fori_loop(0, BATCH // LANES, scan, 0, unroll=2)
    cnt = mcnt[1]
    sel_idx[pl.ds(cnt, LANES)] = jnp.full((LANES,), SENTINEL, jnp.int32)
    ngrp = (cnt + LANES - 1) // LANES

    def extract_rows(c, src_buf):
        def group(u, carry):
            sv = sel_idx[pl.ds(u * LANES, LANES)]
            cv = sv >> 9
            e2 = _match01(cv, c)
            f0 = plsc.all_reduce_ffs(e2 == 1)[0]
            pv = sel_pos[pl.ds(u * LANES, LANES)]

            def has_match(st):
                return st[1] < LANES

            def per_match(st):
                e2c, f = st
                fv = jnp.full((LANES,), 0, jnp.int32) + f
                col = lane_gather(sv, fv)[0] & (SUBC - 1)
                pos = lane_gather(pv, fv)[0]
                mc = mcnt[0]
                slot = mc & 15

                @pl.when(mc >= 16)
                def _():
                    pltpu.make_async_copy(
                        out_hbm.at[pl.ds(0, EMB)],
                        ring.at[pl.ds(0, EMB)],
                        rowsem,
                    ).wait()

                for g in range(EMB // LANES):
                    vals = plsc.load_gather(
                        src_buf,
                        [lane_iota + g * LANES,
                         jnp.full((LANES,), col, jnp.int32)])
                    ring[pl.ds(slot * EMB + g * LANES, LANES)] = vals
                pltpu.async_copy(
                    ring.at[pl.ds(slot * EMB, EMB)],
                    out_hbm.at[pl.ds(pl.multiple_of(pos * EMB, EMB), EMB)],
                    rowsem,
                )
                mcnt[0] = mc + 1
                e2n = e2c * (1 - _match01(lane_iota, f))
                fn = plsc.all_reduce_ffs(e2n == 1)[0]
                return (e2n, fn)

            lax.while_loop(has_match, per_match, (e2, f0))
            return carry

        lax.fori_loop(0, ngrp, group, 0, unroll=False)

    def fetch(c, dst, sem):
        pltpu.async_copy(
            table_t_hbm.at[:, pl.ds(pl.multiple_of(c * SUBC, 128), SUBC)],
            dst, sem)

    def bufwait(dst, sem):
        pltpu.make_async_copy(
            table_t_hbm.at[:, pl.ds(0, SUBC)], dst, sem).wait()

    @pl.when(wid < NSUB)
    def _():
        fetch(wid, buf0, sem0)

    def pair(p, carry):
        ca = wid + (2 * p) * NW
        cb = ca + NW
        cc = cb + NW

        @pl.when(ca < NSUB)
        def _():
            bufwait(buf0, sem0)

            @pl.when(cb < NSUB)
            def _():
                fetch(cb, buf1, sem1)
            extract_rows(ca, buf0)

        @pl.when(cb < NSUB)
        def _():
            bufwait(buf1, sem1)

            @pl.when(cc < NSUB)
            def _():
                fetch(cc, buf0, sem0)
            extract_rows(cb, buf1)
        return carry

    lax.fori_loop(0, (NCHUNK_PER_W + 1) // 2, pair, 0, unroll=False)

    @pl.when(wid == NSUB % NW)
    def _():
        pltpu.sync_copy(table_t_hbm.at[:, pl.ds(NSUB * SUBC, TAIL)], tailbuf)
        extract_rows(jnp.int32(NSUB), tailbuf)

    def drain(r, carry):
        @pl.when(r < jnp.minimum(mcnt[0], 16))
        def _():
            pltpu.make_async_copy(
                out_hbm.at[pl.ds(0, EMB)],
                ring.at[pl.ds(0, EMB)],
                rowsem,
            ).wait()
        return carry

    lax.fori_loop(0, 16, drain, 0, unroll=False)


_gather = pl.kernel(
    _body,
    out_type=jax.ShapeDtypeStruct((BATCH * EMB,), jnp.float32),
    mesh=_MESH,
    compiler_params=pltpu.CompilerParams(needs_layout_passes=False),
    scratch_types=[
        pltpu.VMEM((BATCH,), jnp.int32),
        pltpu.VMEM((SEL_CAP,), jnp.int32),
        pltpu.VMEM((SEL_CAP,), jnp.int32),
        pltpu.VMEM((EMB, SUBC), jnp.float32),
        pltpu.VMEM((EMB, SUBC), jnp.float32),
        pltpu.VMEM((EMB, TAIL), jnp.float32),
        pltpu.VMEM((16 * EMB,), jnp.float32),
        pltpu.SMEM((2,), jnp.int32),
        pltpu.SemaphoreType.DMA,
        pltpu.SemaphoreType.DMA,
        pltpu.SemaphoreType.DMA,
    ],
)


@jax.jit
def kernel(inputs, embedding_table):
    flat = _gather(inputs, embedding_table.T)
    return flat.reshape(BATCH, EMB)
